# baseline (speedup 1.0000x reference)
import functools

import jax
import jax.numpy as jnp
from jax import lax
from jax.experimental import pallas as pl
from jax.experimental.pallas import tpu as pltpu
from jax.experimental.pallas import tpu_sc as plsc

N = 10000
E = 320000
D = 128
DH = D // 2
NC = 2
NS = 16
NW = NC * NS
EP = 327680
EPS = EP // NS
CH = 128
NCH = EPS // CH
RPT = 624
TCB = 1000

ACH = 64
ANCH = (EP // NW) // ACH
APACK = ACH * 48

_f32 = jnp.float32
_i32 = jnp.int32

_MESH = plsc.VectorSubcoreMesh(
    core_axis_name="c", subcore_axis_name="s", num_cores=NC, num_subcores=NS)


N2 = N // 2
RP2 = 312


@functools.partial(
    pl.kernel,
    out_type=jax.ShapeDtypeStruct((NC, N2, D), _f32),
    mesh=_MESH,
    scratch_types=[
        pltpu.VMEM_SHARED((N2, D), _f32),
        pltpu.VMEM_SHARED((N2, D), _f32),
        pltpu.VMEM((3, 2, CH), _i32),
        pltpu.VMEM((3, CH), _f32),
        pltpu.VMEM((3, CH), _i32),
        pltpu.VMEM((2, CH, D), _f32),
        pltpu.SemaphoreType.DMA,
        pltpu.SemaphoreType.DMA,
        pltpu.SemaphoreType.DMA,
        pltpu.SemaphoreType.DMA,
    ],
)
def _spmm_sc(ta, tb, idxr, wr, hr, out, stab, accum, idx_v, w_v, h_v, rows,
             isem, wsem, gsem, ssem):
    c = lax.axis_index("c")
    s = lax.axis_index("s")

    z16 = jnp.zeros((16,), _f32)

    @pl.loop(0, CH)
    def _zero_rows(i):
        for g in range(D // 16):
            rows[0, i, pl.ds(g * 16, 16)] = z16

    base = s * RP2
    for k in range(2):
        pltpu.sync_copy(rows.at[0], accum.at[pl.ds(base + k * CH, CH)])

    @pl.when(s == NS - 1)
    def _zero_tail_full():
        pltpu.sync_copy(rows.at[0, pl.ds(0, 64)],
                        accum.at[pl.ds(base + 2 * CH, 64)])

    @pl.when(s != NS - 1)
    def _zero_tail_part():
        pltpu.sync_copy(rows.at[0, pl.ds(0, RP2 - 2 * CH)],
                        accum.at[pl.ds(base + 2 * CH, RP2 - 2 * CH)])

    @pl.when(c == 0)
    def _stage_a():
        pltpu.sync_copy(ta.at[pl.ds(base, RP2)], stab.at[pl.ds(base, RP2)])

        @pl.when(s == NS - 1)
        def _tail_a():
            pltpu.sync_copy(ta.at[pl.ds(N2 - 8, 8)],
                            stab.at[pl.ds(N2 - 8, 8)])

    @pl.when(c == 1)
    def _stage_b():
        pltpu.sync_copy(tb.at[pl.ds(base, RP2)], stab.at[pl.ds(base, RP2)])

        @pl.when(s == NS - 1)
        def _tail_b():
            pltpu.sync_copy(tb.at[pl.ds(N2 - 8, 8)],
                            stab.at[pl.ds(N2 - 8, 8)])

    plsc.subcore_barrier()

    pltpu.sync_copy(idxr.at[s, 0], idx_v.at[0])
    pltpu.sync_copy(wr.at[s, 0], w_v.at[0])
    pltpu.sync_copy(hr.at[s, 0], h_v.at[0])
    pltpu.async_copy(stab.at[idx_v.at[0, 0]], rows.at[0], gsem)
    pltpu.async_copy(idxr.at[s, 1], idx_v.at[1], isem)
    pltpu.async_copy(wr.at[s, 1], w_v.at[1], wsem)
    pltpu.async_copy(hr.at[s, 1], h_v.at[1], wsem)

    @pl.loop(0, NCH)
    def _chunk(j):
        p = j % 2
        pn = (j + 1) % 2
        b0 = j % 3
        b1 = (j + 1) % 3
        b2 = (j + 2) % 3

        @pl.when(j >= 1)
        def _drain_prev_scatter():
            pltpu.make_async_copy(
                rows.at[pn], accum.at[idx_v.at[b2, 1]], ssem).wait()

        @pl.when(j + 1 < NCH)
        def _start_next_gather():
            pltpu.make_async_copy(idxr.at[s, j + 1], idx_v.at[b1],
                                  isem).wait()
            pltpu.async_copy(stab.at[idx_v.at[b1, 0]], rows.at[pn], gsem)

        @pl.when(j + 2 < NCH)
        def _start_next_idx():
            pltpu.async_copy(idxr.at[s, j + 2], idx_v.at[b2], isem)
            pltpu.async_copy(wr.at[s, j + 2], w_v.at[b2], wsem)
            pltpu.async_copy(hr.at[s, j + 2], h_v.at[b2], wsem)

        pltpu.make_async_copy(stab.at[idx_v.at[b0, 0]], rows.at[p],
                              gsem).wait()

        @pl.when(j >= 1)
        def _wait_w():
            pltpu.make_async_copy(wr.at[s, j], w_v.at[b0], wsem).wait()
            pltpu.make_async_copy(hr.at[s, j], h_v.at[b0], wsem).wait()

        @pl.loop(0, CH // 16)
        def _scale(t):
            wg = w_v[b0, pl.ds(t * 16, 16)]
            hg = h_v[b0, pl.ds(t * 16, 16)]
            for k in range(16):
                e = t * 16 + k
                wv = jnp.full((16,), wg[k], _f32)
                hk = hg[k]
                so = (hk & 1) * DH
                do = ((hk >> 1) & 1) * DH
                zo = DH - do
                vals = [rows[p, e, pl.ds(so + g * 16, 16)]
                        for g in range(DH // 16)]
                for g in range(DH // 16):
                    rows[p, e, pl.ds(do + g * 16, 16)] = vals[g] * wv
                for g in range(DH // 16):
                    rows[p, e, pl.ds(zo + g * 16, 16)] = z16

        pltpu.async_copy(rows.at[p], accum.at[idx_v.at[b0, 1]], ssem,
                         add=True)

    pltpu.make_async_copy(
        rows.at[(NCH - 1) % 2],
        accum.at[idx_v.at[(NCH - 1) % 3, 1]], ssem).wait()

    plsc.subcore_barrier()
    for k in range(2):
        pltpu.sync_copy(accum.at[pl.ds(base + k * CH, CH)],
                        out.at[c, pl.ds(base + k * CH, CH)])

    @pl.when(s == NS - 1)
    def _write_tail_full():
        pltpu.sync_copy(accum.at[pl.ds(base + 2 * CH, 64)],
                        out.at[c, pl.ds(base + 2 * CH, 64)])

    @pl.when(s != NS - 1)
    def _write_tail_part():
        pltpu.sync_copy(accum.at[pl.ds(base + 2 * CH, RP2 - 2 * CH)],
                        out.at[c, pl.ds(base + 2 * CH, RP2 - 2 * CH)])


@functools.partial(
    pl.kernel,
    out_type=jax.ShapeDtypeStruct((NW, ANCH, APACK), _f32),
    mesh=_MESH,
    scratch_types=[
        pltpu.VMEM_SHARED((N, D), _f32),
        pltpu.VMEM((3, 2, ACH), _i32),
        pltpu.VMEM((2, ACH, D), _f32),
        pltpu.VMEM((2, ACH, D), _f32),
        pltpu.VMEM((2, APACK), _f32),
        pltpu.SemaphoreType.DMA,
        pltpu.SemaphoreType.DMA,
        pltpu.SemaphoreType.DMA,
    ],
)
def _attn_sc(gtab, idxr, part_o, stab, idx_v, arows, brows, pall,
             isem, gsem, osem):
    c = lax.axis_index("c")
    s = lax.axis_index("s")
    wid = s * NC + c

    base = s * RPT
    pltpu.sync_copy(gtab.at[pl.ds(base, RPT)], stab.at[pl.ds(base, RPT)])

    @pl.when(s == NS - 1)
    def _stage_tail():
        pltpu.sync_copy(gtab.at[pl.ds(N - 16, 16)],
                        stab.at[pl.ds(N - 16, 16)])

    plsc.subcore_barrier()

    pltpu.sync_copy(idxr.at[wid, 0], idx_v.at[0])
    pltpu.async_copy(stab.at[idx_v.at[0, 0]], arows.at[0], gsem)
    pltpu.async_copy(stab.at[idx_v.at[0, 1]], brows.at[0], gsem)
    pltpu.async_copy(idxr.at[wid, 1], idx_v.at[1], isem)

    @pl.loop(0, ANCH)
    def _chunk(j):
        p = j % 2
        pn = (j + 1) % 2
        b0 = j % 3
        b1 = (j + 1) % 3
        b2 = (j + 2) % 3

        @pl.when(j >= 1)
        def _drain_prev_out():
            pltpu.make_async_copy(pall.at[pn], part_o.at[wid, j - 1],
                                  osem).wait()

        @pl.when(j + 1 < ANCH)
        def _start_next_gather():
            pltpu.make_async_copy(idxr.at[wid, j + 1], idx_v.at[b1],
                                  isem).wait()
            pltpu.async_copy(stab.at[idx_v.at[b1, 0]], arows.at[pn], gsem)
            pltpu.async_copy(stab.at[idx_v.at[b1, 1]], brows.at[pn], gsem)

        @pl.when(j + 2 < ANCH)
        def _start_next_idx():
            pltpu.async_copy(idxr.at[wid, j + 2], idx_v.at[b2], isem)

        pltpu.make_async_copy(stab.at[idx_v.at[b0, 0]], arows.at[p],
                              gsem).wait()
        pltpu.make_async_copy(stab.at[idx_v.at[b0, 1]], brows.at[p],
                              gsem).wait()

        @pl.loop(0, ACH, unroll=2)
        def _edge(e):
            a = arows[p, e, pl.ds(0, 16)]
            b = brows[p, e, pl.ds(0, 16)]
            pab = a * b
            paa = a * a
            pbb = b * b
            for g in range(1, D // 16):
                a = arows[p, e, pl.ds(g * 16, 16)]
                b = brows[p, e, pl.ds(g * 16, 16)]
                pab = pab + a * b
                paa = paa + a * a
                pbb = pbb + b * b
            pall[p, pl.ds(e * 48, 16)] = pab
            pall[p, pl.ds(e * 48 + 16, 16)] = paa
            pall[p, pl.ds(e * 48 + 32, 16)] = pbb

        pltpu.async_copy(pall.at[p], part_o.at[wid, j], osem)

    pltpu.make_async_copy(pall.at[(ANCH - 1) % 2],
                          part_o.at[wid, ANCH - 1], osem).wait()


def _dotT(a, w):
    return lax.dot_general(a, w, (((1,), (1,)), ((), ())),
                           preferred_element_type=_f32)


_blk = pl.BlockSpec((TCB, D), lambda i: (i, 0))
_hblk = pl.BlockSpec((TCB, DH), lambda i: (i, 0))
_pblk = pl.BlockSpec((NC, TCB, DH), lambda i: (0, i, 0))
_wspec = pl.BlockSpec((D, D), lambda i: (0, 0))
_bspec = pl.BlockSpec((D,), lambda i: (0,))
_tspec = pl.BlockSpec(memory_space=pltpu.SMEM)


def _tc_in(x, W_in, b_in, W1, b1):
    def body(x_r, wi_r, bi_r, w1_r, b1_r, h_r, la_r, lb_r):
        h = _dotT(x_r[...], wi_r[...]) + bi_r[...][None, :]
        h_r[...] = h
        lin1 = _dotT(h, w1_r[...]) + b1_r[...][None, :]
        la_r[...] = lin1[:, :DH]
        lb_r[...] = lin1[:, DH:]

    return pl.pallas_call(
        body,
        grid=(N // TCB,),
        in_specs=[_blk, _wspec, _bspec, _wspec, _bspec],
        out_specs=[_blk, _hblk, _hblk],
        out_shape=[jax.ShapeDtypeStruct((N, D), _f32),
                   jax.ShapeDtypeStruct((N, DH), _f32),
                   jax.ShapeDtypeStruct((N, DH), _f32)],
    )(x, W_in, b_in, W1, b1)


def _tc_layer(pa, pb, h, W2, b2, temp):
    def body(pa_r, pb_r, h_r, w2_r, b2_r, t_r, la_r, lb_r, hidp_r):
        cur = jnp.maximum(
            jnp.concatenate([pa_r[...], pb_r[...]], axis=1), 0.0)
        lin2 = _dotT(cur, w2_r[...]) + b2_r[...][None, :]
        la_r[...] = lin2[:, :DH]
        lb_r[...] = lin2[:, DH:]
        hidp_r[...] = h_r[...] * t_r[0] + cur * t_r[1]

    return pl.pallas_call(
        body,
        grid=(N // TCB,),
        in_specs=[_hblk, _hblk, _blk, _wspec, _bspec, _tspec],
        out_specs=[_hblk, _hblk, _blk],
        out_shape=[jax.ShapeDtypeStruct((N, DH), _f32),
                   jax.ShapeDtypeStruct((N, DH), _f32),
                   jax.ShapeDtypeStruct((N, D), _f32)],
    )(pa, pb, h, W2, b2, temp)


def _tc_extract(pa, pb, hidp, temp, We1, be1, We2, be2):
    def body(pa_r, pb_r, hidp_r, t_r, we1_r, be1_r, we2_r, be2_r, g_r):
        cur2 = jnp.maximum(
            jnp.concatenate([pa_r[...], pb_r[...]], axis=1), 0.0)
        hgnn = hidp_r[...] + cur2 * t_r[2]
        t1 = jnp.maximum(_dotT(hgnn, we1_r[...]) + be1_r[...][None, :], 0.0)
        g_r[...] = _dotT(t1, we2_r[...]) + be2_r[...][None, :]

    return pl.pallas_call(
        body,
        grid=(N // TCB,),
        in_specs=[_hblk, _hblk, _blk, _tspec, _wspec, _bspec, _wspec,
                  _bspec],
        out_specs=_blk,
        out_shape=jax.ShapeDtypeStruct((N, D), _f32),
    )(pa, pb, hidp, temp, We1, be1, We2, be2)


def _tc_attnw(wf, part):
    TB = 8192

    def body(w_r, p_r, o_r):
        r = lax.broadcasted_iota(_i32, (48, 3), 0)
        cc = lax.broadcasted_iota(_i32, (48, 3), 1)
        sel = ((r // 16) == cc).astype(_f32)
        sums = jnp.dot(p_r[...], sel, preferred_element_type=_f32)
        num = sums[:, 0]
        den = jnp.maximum(jnp.sqrt(sums[:, 1] * sums[:, 2]), 1e-8)
        o_r[...] = w_r[...] * (num / den)

    return pl.pallas_call(
        body,
        grid=(EP // TB,),
        in_specs=[pl.BlockSpec((TB,), lambda i: (i,)),
                  pl.BlockSpec((TB, 48), lambda i: (i, 0))],
        out_specs=pl.BlockSpec((TB,), lambda i: (i,)),
        out_shape=jax.ShapeDtypeStruct((EP,), _f32),
    )(wf, part)


def _tc_out(pa, pb, hidp2, temp, W_out, b_out):
    def body(pa_r, pb_r, hidp_r, t_r, wo_r, bo_r, o_r):
        cur = jnp.maximum(
            jnp.concatenate([pa_r[...], pb_r[...]], axis=1), 0.0)
        hgnn2 = hidp_r[...] + cur * t_r[2]
        o_r[...] = _dotT(hgnn2, wo_r[...]) + bo_r[...][None, :]

    return pl.pallas_call(
        body,
        grid=(N // TCB,),
        in_specs=[_hblk, _hblk, _blk, _tspec, _wspec, _bspec],
        out_specs=_blk,
        out_shape=jax.ShapeDtypeStruct((N, D), _f32),
    )(pa, pb, hidp2, temp, W_out, b_out)


def kernel(x, edge_index, edge_w, W_in, b_in, W1, b1, W2, b2,
           We1, be1, We2, be2, W_out, b_out, temp):
    padi = jnp.zeros((EP - E,), _i32)
    srcp = jnp.concatenate([edge_index[0], padi])
    dstp = jnp.concatenate([edge_index[1], padi])
    padf = jnp.zeros((EP - E,), _f32)
    wp = jnp.concatenate([edge_w, padf])

    qsrc = (srcp // 2).reshape(NS, NCH, CH)
    qdst = (dstp // 2).reshape(NS, NCH, CH)
    idxr = jnp.stack([qsrc, qdst], axis=2)
    hr = ((srcp & 1) + 2 * (dstp & 1)).reshape(NS, NCH, CH)
    wr = wp.reshape(NS, NCH, CH)
    aidxr = jnp.stack([srcp.reshape(NW, ANCH, ACH),
                       dstp.reshape(NW, ANCH, ACH)], axis=2)

    h, l1a, l1b = _tc_in(x, W_in, b_in, W1, b1)
    p1 = _spmm_sc(l1a.reshape(N2, D), l1b.reshape(N2, D), idxr, wr, hr)
    p1a, p1b = p1[0].reshape(N, DH), p1[1].reshape(N, DH)
    l2a, l2b, hidp = _tc_layer(p1a, p1b, h, W2, b2, temp)
    p2 = _spmm_sc(l2a.reshape(N2, D), l2b.reshape(N2, D), idxr, wr, hr)
    g = _tc_extract(p2[0].reshape(N, DH), p2[1].reshape(N, DH),
                    hidp, temp, We1, be1, We2, be2)
    part = _attn_sc(g, aidxr)
    w2 = _tc_attnw(wp, part.reshape(EP, 48))
    w2r = w2.reshape(NS, NCH, CH)

    p3 = _spmm_sc(l1a.reshape(N2, D), l1b.reshape(N2, D), idxr, w2r, hr)
    l2a2, l2b2, hidp2 = _tc_layer(p3[0].reshape(N, DH),
                                  p3[1].reshape(N, DH), h, W2, b2, temp)
    p4 = _spmm_sc(l2a2.reshape(N2, D), l2b2.reshape(N2, D), idxr, w2r, hr)
    return _tc_out(p4[0].reshape(N, DH), p4[1].reshape(N, DH),
                   hidp2, temp, W_out, b_out)
